# Initial kernel scaffold; baseline (speedup 1.0000x reference)
import jax
import jax.numpy as jnp
from jax.experimental import pallas as pl


def kernel(x, edge_index, cand_edges, W, att_src, att_dst, bias, W1, b1, W2, b2):
    raise NotImplementedError("write your pallas kernel here")


# speedup vs baseline: 12.4343x; 12.4343x over previous
import functools

import jax
import jax.numpy as jnp
from jax.experimental import pallas as pl


def _mlp_body(pair_ref, w1_ref, b1_ref, w2_ref, b2_ref, out_ref):
    pair = pair_ref[...]
    hid = jnp.maximum(pair @ w1_ref[...] + b1_ref[...][None, :], 0.0)
    logit = hid @ w2_ref[...] + b2_ref[...][None, :]
    out_ref[...] = jax.nn.sigmoid(logit)


def _mlp_pallas(pair, W1, b1, W2, b2):
    NC = pair.shape[0]
    BLK = 5000
    grid = NC // BLK
    out2d = pl.pallas_call(
        _mlp_body,
        grid=(grid,),
        in_specs=[
            pl.BlockSpec((BLK, 96), lambda i: (i, 0)),
            pl.BlockSpec((96, 128), lambda i: (0, 0)),
            pl.BlockSpec((128,), lambda i: (0,)),
            pl.BlockSpec((128, 1), lambda i: (0, 0)),
            pl.BlockSpec((1,), lambda i: (0,)),
        ],
        out_specs=pl.BlockSpec((BLK, 1), lambda i: (i, 0)),
        out_shape=jax.ShapeDtypeStruct((NC, 1), jnp.float32),
    )(pair, W1, b1, W2, b2)
    return out2d.reshape(NC)


def kernel(x, edge_index, cand_edges, W, att_src, att_dst, bias, W1, b1, W2, b2):
    N, D = x.shape
    H, C = att_src.shape
    Wh = W.reshape(D, H, C)
    As = jnp.einsum("dhc,hc->dh", Wh, att_src)
    Ad = jnp.einsum("dhc,hc->dh", Wh, att_dst)
    a_src = x @ As
    a_dst = x @ Ad
    src, dst = edge_index[0], edge_index[1]
    alpha = jax.nn.leaky_relu(a_src[src] + a_dst[dst], 0.2)
    p = jnp.exp(alpha)
    xa = jnp.concatenate([jnp.ones((N, 1), jnp.float32), x], axis=1)
    contrib = (p[:, :, None] * xa[src][:, None, :]).reshape(-1, 8 * H)
    S = jax.ops.segment_sum(contrib, dst, num_segments=N)
    p_self = jnp.exp(jax.nn.leaky_relu(a_src + a_dst, 0.2))
    S = S + (p_self[:, :, None] * xa[:, None, :]).reshape(N, 8 * H)
    S = S.reshape(N, H, 8)
    denom = S[:, :, 0]
    Snorm = S[:, :, 1:] / (denom[:, :, None] + 1e-16)
    out = jnp.einsum("nhd,dhc->nc", Snorm, Wh) / H + bias[None, :]
    h = jax.nn.elu(out)
    u = h[cand_edges[:, 0]]
    v = h[cand_edges[:, 1]]
    pair = jnp.concatenate([u, v, jnp.abs(u - v)], axis=1)
    return _mlp_pallas(pair, W1, b1, W2, b2)


# speedup vs baseline: 95.7113x; 7.6974x over previous
import functools

import jax
import jax.numpy as jnp
from jax import lax
from jax.experimental import pallas as pl
from jax.experimental.pallas import tpu as pltpu, tpu_sc as plsc

N = 100000
E = 3200000
NCAND = 500000
NH = 8
NCH = 32
NCORES = 2
NSUB = 16
NW = NCORES * NSUB

SUB = 80
NSUBBLK = 8
BLK = SUB * NSUBBLK
NBTOT = E // BLK
NPAIR = BLK // 2

NCP = NW * 123 * 128
CPT = NCP // NW
CIT = CPT // 128

_SC_MESH = plsc.VectorSubcoreMesh(core_axis_name="c", subcore_axis_name="s")
_SC_PARAMS = pltpu.CompilerParams(use_tc_tiling_on_sc=False)


def _prep_body(x_ref, aa_ref, as2_ref, ad2_ref, xa_ref, init_ref):
    xb = x_ref[...]
    asd = xb @ aa_ref[...]
    a_s = asd[:, :8]
    a_d = asd[:, 8:]
    as2_ref[...] = jnp.concatenate([a_s, a_s], axis=1)
    ad2_ref[...] = jnp.concatenate([a_d, a_d], axis=1)
    bn = xb.shape[0]
    xab = jnp.concatenate([jnp.ones((bn, 1), jnp.float32), xb], axis=1)
    xa_ref[...] = jnp.concatenate([xab, xab], axis=1)
    al = a_s + a_d
    al = jnp.where(al >= 0, al, 0.2 * al)
    ps = jnp.exp(al)
    init_ref[...] = ps[:, :, None] * xab[:, None, :]


def _prep(x, aa):
    BN = 1000
    g = N // BN
    return pl.pallas_call(
        _prep_body,
        grid=(g,),
        in_specs=[
            pl.BlockSpec((BN, 7), lambda i: (i, 0)),
            pl.BlockSpec((7, 16), lambda i: (0, 0)),
        ],
        out_specs=[
            pl.BlockSpec((BN, 16), lambda i: (i, 0)),
            pl.BlockSpec((BN, 16), lambda i: (i, 0)),
            pl.BlockSpec((BN, 16), lambda i: (i, 0)),
            pl.BlockSpec((BN, 8, 8), lambda i: (i, 0, 0)),
        ],
        out_shape=[
            jax.ShapeDtypeStruct((N, 16), jnp.float32),
            jax.ShapeDtypeStruct((N, 16), jnp.float32),
            jax.ShapeDtypeStruct((N, 16), jnp.float32),
            jax.ShapeDtypeStruct((N, 8, 8), jnp.float32),
        ],
    )(x, aa)


def _edge_a_body(src_ref, dst_ref, as2_ref, ad2_ref, xa_ref, p_ref, c_ref,
                 sidx, didx, sa, da, cg, pb, sem):
    wid = lax.axis_index("s") * NCORES + lax.axis_index("c")

    def iter_body(j, carry):
        b = j * NW + wid
        rowb = b * NSUBBLK
        base = b * BLK
        pltpu.sync_copy(src_ref.at[pl.ds(rowb, NSUBBLK)], sidx)
        pltpu.sync_copy(dst_ref.at[pl.ds(rowb, NSUBBLK)], didx)
        cps = []
        for k in range(NSUBBLK):
            cps.append(pltpu.async_copy(
                as2_ref.at[sidx.at[k]], sa.at[pl.ds(k * SUB, SUB)], sem))
            cps.append(pltpu.async_copy(
                ad2_ref.at[didx.at[k]], da.at[pl.ds(k * SUB, SUB)], sem))
            cps.append(pltpu.async_copy(
                xa_ref.at[sidx.at[k]], cg.at[pl.ds(k * SUB, SUB)], sem))
        for cp in cps:
            cp.wait()

        def pair_body(i, carry2):
            e0 = 2 * i
            v0 = sa[e0] + da[e0]
            v0 = jnp.where(v0 >= 0, v0, 0.2 * v0)
            pb[e0] = jnp.exp(v0)
            e1 = 2 * i + 1
            v1 = sa[e1] + da[e1]
            v1 = jnp.where(v1 >= 0, v1, 0.2 * v1)
            pb[e1] = jnp.exp(v1)
            return carry2
        lax.fori_loop(0, NPAIR, pair_body, 0)

        pltpu.sync_copy(pb, p_ref.at[pl.ds(base, BLK)])
        pltpu.sync_copy(cg, c_ref.at[pl.ds(base, BLK)])
        return carry
    nit = (NBTOT // NW) + jnp.where(wid < NBTOT % NW, 1, 0)
    lax.fori_loop(0, nit, iter_body, 0)


def _edge_a(src2d, dst2d, as2, ad2, xa):
    k = pl.kernel(
        _edge_a_body,
        out_type=[
            jax.ShapeDtypeStruct((E, 16), jnp.float32),
            jax.ShapeDtypeStruct((E, 16), jnp.float32),
        ],
        mesh=_SC_MESH,
        compiler_params=_SC_PARAMS,
        scratch_types=[
            pltpu.VMEM((NSUBBLK, SUB), jnp.int32),
            pltpu.VMEM((NSUBBLK, SUB), jnp.int32),
            pltpu.VMEM((BLK, 16), jnp.float32),
            pltpu.VMEM((BLK, 16), jnp.float32),
            pltpu.VMEM((BLK, 16), jnp.float32),
            pltpu.VMEM((BLK, 16), jnp.float32),
            pltpu.SemaphoreType.DMA,
        ],
    )
    return k(src2d, dst2d, as2, ad2, xa)


def _edge_b_body(dst_ref, p_ref, c_ref, init_ref, zin_ref, spart_ref,
                 didx, pv, cb, s2, sem):
    cid = lax.axis_index("c")
    sid = lax.axis_index("s")
    wid = sid * NCORES + cid
    low8 = lax.iota(jnp.int32, 16) < 8
    CHUNK = 6256
    s_lo = sid * CHUNK
    NSL = N - 15 * CHUNK
    nit = (NBTOT // NW) + jnp.where(wid < NBTOT % NW, 1, 0)

    for g in range(4):
        h0 = 2 * g

        def mk_init(L, from_init, g=g):
            def f():
                if from_init:
                    pltpu.sync_copy(init_ref.at[g, pl.ds(s_lo, L)],
                                    s2.at[pl.ds(s_lo, L)])
                else:
                    pltpu.sync_copy(zin_ref.at[pl.ds(s_lo, L)],
                                    s2.at[pl.ds(s_lo, L)])
            return f

        pl.when((cid == 0) & (sid < 15))(mk_init(CHUNK, True))
        pl.when((cid == 0) & (sid == 15))(mk_init(NSL, True))
        pl.when((cid == 1) & (sid < 15))(mk_init(CHUNK, False))
        pl.when((cid == 1) & (sid == 15))(mk_init(NSL, False))

        plsc.subcore_barrier()

        def iter_body(j, carry, h0=h0):
            b = j * NW + wid
            rowb = b * NSUBBLK
            base = b * BLK
            pltpu.sync_copy(dst_ref.at[pl.ds(rowb, NSUBBLK)], didx)
            pltpu.sync_copy(c_ref.at[pl.ds(base, BLK)], cb)
            pltpu.sync_copy(p_ref.at[pl.ds(base, BLK)], pv)

            def pair_body(i, carry2):
                e0 = 2 * i
                e1 = 2 * i + 1
                pr0 = pv[e0]
                pr1 = pv[e1]
                pe0 = jnp.where(low8, pr0[h0], pr0[h0 + 1])
                pe1 = jnp.where(low8, pr1[h0], pr1[h0 + 1])
                cb[e0] = cb[e0] * pe0
                cb[e1] = cb[e1] * pe1
                return carry2
            lax.fori_loop(0, NPAIR, pair_body, 0)

            cps = []
            for k in range(NSUBBLK):
                cps.append(pltpu.async_copy(
                    cb.at[pl.ds(k * SUB, SUB)], s2.at[didx.at[k]], sem,
                    add=True))
            for cp in cps:
                cp.wait()
            return carry
        lax.fori_loop(0, nit, iter_body, 0)

        plsc.subcore_barrier()

        def mk_out(L, g=g):
            def f():
                pltpu.sync_copy(s2.at[pl.ds(s_lo, L)],
                                spart_ref.at[g, pl.ds(cid * N + s_lo, L)])
            return f

        pl.when(sid < 15)(mk_out(CHUNK))
        pl.when(sid == 15)(mk_out(NSL))


def _edge_b(dst2d, p_flat, c_flat, init4, zin):
    k = pl.kernel(
        _edge_b_body,
        out_type=[jax.ShapeDtypeStruct((4, NCORES * N, 16), jnp.float32)],
        mesh=_SC_MESH,
        compiler_params=_SC_PARAMS,
        scratch_types=[
            pltpu.VMEM((NSUBBLK, SUB), jnp.int32),
            pltpu.VMEM((BLK, 16), jnp.float32),
            pltpu.VMEM((BLK, 16), jnp.float32),
            pltpu.VMEM_SHARED((N, 16), jnp.float32),
            pltpu.SemaphoreType.DMA,
        ],
    )
    (spart,) = k(dst2d, p_flat, c_flat, init4, zin)
    return spart


def _comb_body(sp_ref, wh_ref, b_ref, out_ref):
    sp = sp_ref[...]
    s = sp[:, 0] + sp[:, 1]
    bn = s.shape[1]
    acc = jnp.zeros((bn, NCH), jnp.float32)
    for g in range(4):
        for j in range(2):
            sh = s[g, :, 8 * j:8 * j + 8]
            r = sh[:, 1:8] / (sh[:, 0:1] + 1e-16)
            acc = acc + r @ wh_ref[2 * g + j]
    o = acc * (1.0 / NH) + b_ref[...][None, :]
    out_ref[...] = jnp.where(o > 0, o, jnp.exp(o) - 1.0)


def _combine(spart, wh_t, bias):
    BN = 1000
    g = N // BN
    return pl.pallas_call(
        _comb_body,
        grid=(g,),
        in_specs=[
            pl.BlockSpec((4, NCORES, BN, 16), lambda i: (0, 0, i, 0)),
            pl.BlockSpec((NH, 7, NCH), lambda i: (0, 0, 0)),
            pl.BlockSpec((NCH,), lambda i: (0,)),
        ],
        out_specs=pl.BlockSpec((BN, NCH), lambda i: (i, 0)),
        out_shape=jax.ShapeDtypeStruct((N, NCH), jnp.float32),
    )(spart, wh_t, bias)


def _pair_body(cu_ref, cv_ref, h_ref, u_ref, v_ref, iu, iv, bu, bv, sem):
    wid = lax.axis_index("s") * NCORES + lax.axis_index("c")

    def iter_body(j, carry):
        base = wid * CPT + j * 128
        pltpu.sync_copy(cu_ref.at[pl.ds(base, 128)], iu)
        pltpu.sync_copy(cv_ref.at[pl.ds(base, 128)], iv)
        a = pltpu.async_copy(h_ref.at[iu], bu, sem)
        b = pltpu.async_copy(h_ref.at[iv], bv, sem)
        a.wait()
        b.wait()
        pltpu.sync_copy(bu, u_ref.at[pl.ds(base, 128)])
        pltpu.sync_copy(bv, v_ref.at[pl.ds(base, 128)])
        return carry
    lax.fori_loop(0, CIT, iter_body, 0)


def _pair_gather(cu, cv, hemb):
    k = pl.kernel(
        _pair_body,
        out_type=[
            jax.ShapeDtypeStruct((NCP, NCH), jnp.float32),
            jax.ShapeDtypeStruct((NCP, NCH), jnp.float32),
        ],
        mesh=_SC_MESH,
        compiler_params=_SC_PARAMS,
        scratch_types=[
            pltpu.VMEM((128,), jnp.int32),
            pltpu.VMEM((128,), jnp.int32),
            pltpu.VMEM((128, NCH), jnp.float32),
            pltpu.VMEM((128, NCH), jnp.float32),
            pltpu.SemaphoreType.DMA,
        ],
    )
    return k(cu, cv, hemb)


def _mlp_body(u_ref, v_ref, w1_ref, b1_ref, w2_ref, b2_ref, out_ref):
    u = u_ref[...]
    v = v_ref[...]
    pair = jnp.concatenate([u, v, jnp.abs(u - v)], axis=1)
    hid = jnp.maximum(pair @ w1_ref[...] + b1_ref[...][None, :], 0.0)
    logit = hid @ w2_ref[...] + b2_ref[...][None, :]
    out_ref[...] = jax.nn.sigmoid(logit)


def _mlp(u, v, W1, b1, W2, b2):
    BN = 5000
    g = NCAND // BN
    out2d = pl.pallas_call(
        _mlp_body,
        grid=(g,),
        in_specs=[
            pl.BlockSpec((BN, NCH), lambda i: (i, 0)),
            pl.BlockSpec((BN, NCH), lambda i: (i, 0)),
            pl.BlockSpec((96, 128), lambda i: (0, 0)),
            pl.BlockSpec((128,), lambda i: (0,)),
            pl.BlockSpec((128, 1), lambda i: (0, 0)),
            pl.BlockSpec((1,), lambda i: (0,)),
        ],
        out_specs=pl.BlockSpec((BN, 1), lambda i: (i, 0)),
        out_shape=jax.ShapeDtypeStruct((NCAND, 1), jnp.float32),
    )(u, v, W1, b1, W2, b2)
    return out2d.reshape(NCAND)


def kernel(x, edge_index, cand_edges, W, att_src, att_dst, bias, W1, b1, W2, b2):
    D = x.shape[1]
    wh = W.reshape(D, NH, NCH)
    a_s = jnp.einsum("dhc,hc->dh", wh, att_src)
    a_d = jnp.einsum("dhc,hc->dh", wh, att_dst)
    aa = jnp.concatenate([a_s, a_d], axis=1)
    wh_t = wh.transpose(1, 0, 2)

    as2, ad2, xa, init_s = _prep(x, aa)
    init4 = init_s.transpose(1, 0, 2).reshape(4, 2, N, 8)
    init4 = init4.transpose(0, 2, 1, 3).reshape(4, N, 16)

    src2d = edge_index[0].reshape(E // SUB, SUB)
    dst2d = edge_index[1].reshape(E // SUB, SUB)
    zin = jnp.zeros((N, 16), jnp.float32)

    p_arr, c_arr = _edge_a(src2d, dst2d, as2, ad2, xa)
    spart = _edge_b(dst2d, p_arr, c_arr, init4, zin)
    hemb = _combine(spart.reshape(4, NCORES, N, 16), wh_t, bias)

    cu = jnp.pad(cand_edges[:, 0], (0, NCP - NCAND))
    cv = jnp.pad(cand_edges[:, 1], (0, NCP - NCAND))
    u, v = _pair_gather(cu, cv, hemb)
    return _mlp(u[:NCAND], v[:NCAND], W1, b1, W2, b2)


# speedup vs baseline: 110.2500x; 1.1519x over previous
import functools

import jax
import jax.numpy as jnp
from jax import lax
from jax.experimental import pallas as pl
from jax.experimental.pallas import tpu as pltpu, tpu_sc as plsc

N = 100000
E = 3200000
NCAND = 500000
NH = 8
NCH = 32
NCORES = 2
NSUB = 16
NW = NCORES * NSUB

SUB = 80
NSUBBLK = 8
BLK = SUB * NSUBBLK
NBTOT = E // BLK
NPAIR = BLK // 2

NCP = NW * 123 * 128
CPT = NCP // NW
CIT = CPT // 128

_SC_MESH = plsc.VectorSubcoreMesh(core_axis_name="c", subcore_axis_name="s")
_SC_PARAMS = pltpu.CompilerParams(use_tc_tiling_on_sc=False)


def _prep_body(x_ref, aa_ref, as2_ref, ad2_ref, xa_ref, init_ref):
    xb = x_ref[...]
    asd = xb @ aa_ref[...]
    a_s = asd[:, :8]
    a_d = asd[:, 8:]
    as2_ref[...] = jnp.concatenate([a_s, a_s], axis=1)
    ad2_ref[...] = jnp.concatenate([a_d, a_d], axis=1)
    bn = xb.shape[0]
    xab = jnp.concatenate([jnp.ones((bn, 1), jnp.float32), xb], axis=1)
    xa_ref[...] = jnp.concatenate([xab, xab], axis=1)
    al = a_s + a_d
    al = jnp.where(al >= 0, al, 0.2 * al)
    ps = jnp.exp(al)
    init_ref[...] = ps[:, :, None] * xab[:, None, :]


def _prep(x, aa):
    BN = 1000
    g = N // BN
    return pl.pallas_call(
        _prep_body,
        grid=(g,),
        in_specs=[
            pl.BlockSpec((BN, 7), lambda i: (i, 0)),
            pl.BlockSpec((7, 16), lambda i: (0, 0)),
        ],
        out_specs=[
            pl.BlockSpec((BN, 16), lambda i: (i, 0)),
            pl.BlockSpec((BN, 16), lambda i: (i, 0)),
            pl.BlockSpec((BN, 16), lambda i: (i, 0)),
            pl.BlockSpec((BN, 8, 8), lambda i: (i, 0, 0)),
        ],
        out_shape=[
            jax.ShapeDtypeStruct((N, 16), jnp.float32),
            jax.ShapeDtypeStruct((N, 16), jnp.float32),
            jax.ShapeDtypeStruct((N, 16), jnp.float32),
            jax.ShapeDtypeStruct((N, 8, 8), jnp.float32),
        ],
    )(x, aa)


def _edge_a_body(src_ref, dst_ref, as2_ref, ad2_ref, xa_ref, p_ref, c_ref,
                 sidx, didx, sa, da, cg, pb, sem):
    wid = lax.axis_index("s") * NCORES + lax.axis_index("c")

    def iter_body(j, carry):
        b = j * NW + wid
        rowb = b * NSUBBLK
        base = b * BLK
        c1 = pltpu.async_copy(src_ref.at[pl.ds(rowb, NSUBBLK)], sidx, sem)
        c2 = pltpu.async_copy(dst_ref.at[pl.ds(rowb, NSUBBLK)], didx, sem)
        c1.wait()
        c2.wait()
        cps = []
        for k in range(NSUBBLK):
            cps.append(pltpu.async_copy(
                as2_ref.at[sidx.at[k]], sa.at[pl.ds(k * SUB, SUB)], sem))
            cps.append(pltpu.async_copy(
                ad2_ref.at[didx.at[k]], da.at[pl.ds(k * SUB, SUB)], sem))
            cps.append(pltpu.async_copy(
                xa_ref.at[sidx.at[k]], cg.at[pl.ds(k * SUB, SUB)], sem))
        for cp in cps:
            cp.wait()

        def pair_body(i, carry2):
            e0 = 2 * i
            v0 = sa[e0] + da[e0]
            v0 = jnp.where(v0 >= 0, v0, 0.2 * v0)
            pb[e0] = jnp.exp(v0)
            e1 = 2 * i + 1
            v1 = sa[e1] + da[e1]
            v1 = jnp.where(v1 >= 0, v1, 0.2 * v1)
            pb[e1] = jnp.exp(v1)
            return carry2
        lax.fori_loop(0, NPAIR, pair_body, 0)

        o1 = pltpu.async_copy(pb, p_ref.at[pl.ds(base, BLK)], sem)
        o2 = pltpu.async_copy(cg, c_ref.at[pl.ds(base, BLK)], sem)
        o1.wait()
        o2.wait()
        return carry
    nit = (NBTOT // NW) + jnp.where(wid < NBTOT % NW, 1, 0)
    lax.fori_loop(0, nit, iter_body, 0)


def _edge_a(src2d, dst2d, as2, ad2, xa):
    k = pl.kernel(
        _edge_a_body,
        out_type=[
            jax.ShapeDtypeStruct((E, 16), jnp.float32),
            jax.ShapeDtypeStruct((E, 16), jnp.float32),
        ],
        mesh=_SC_MESH,
        compiler_params=_SC_PARAMS,
        scratch_types=[
            pltpu.VMEM((NSUBBLK, SUB), jnp.int32),
            pltpu.VMEM((NSUBBLK, SUB), jnp.int32),
            pltpu.VMEM((BLK, 16), jnp.float32),
            pltpu.VMEM((BLK, 16), jnp.float32),
            pltpu.VMEM((BLK, 16), jnp.float32),
            pltpu.VMEM((BLK, 16), jnp.float32),
            pltpu.SemaphoreType.DMA,
        ],
    )
    return k(src2d, dst2d, as2, ad2, xa)


def _edge_b_body(dst_ref, p_ref, c_ref, init_ref, zin_ref, spart_ref,
                 didx, pv, cb, s2, sem):
    cid = lax.axis_index("c")
    sid = lax.axis_index("s")
    wid = sid * NCORES + cid
    low8 = lax.iota(jnp.int32, 16) < 8
    CHUNK = 6256
    s_lo = sid * CHUNK
    NSL = N - 15 * CHUNK
    nit = (NBTOT // NW) + jnp.where(wid < NBTOT % NW, 1, 0)

    for g in range(4):
        h0 = 2 * g

        def mk_init(L, from_init, g=g):
            def f():
                if from_init:
                    pltpu.sync_copy(init_ref.at[g, pl.ds(s_lo, L)],
                                    s2.at[pl.ds(s_lo, L)])
                else:
                    pltpu.sync_copy(zin_ref.at[pl.ds(s_lo, L)],
                                    s2.at[pl.ds(s_lo, L)])
            return f

        pl.when((cid == 0) & (sid < 15))(mk_init(CHUNK, True))
        pl.when((cid == 0) & (sid == 15))(mk_init(NSL, True))
        pl.when((cid == 1) & (sid < 15))(mk_init(CHUNK, False))
        pl.when((cid == 1) & (sid == 15))(mk_init(NSL, False))

        plsc.subcore_barrier()

        def iter_body(j, carry, h0=h0):
            b = j * NW + wid
            rowb = b * NSUBBLK
            base = b * BLK
            c1 = pltpu.async_copy(dst_ref.at[pl.ds(rowb, NSUBBLK)], didx, sem)
            c2 = pltpu.async_copy(c_ref.at[pl.ds(base, BLK)], cb, sem)
            c3 = pltpu.async_copy(p_ref.at[pl.ds(base, BLK)], pv, sem)
            c1.wait()
            c2.wait()
            c3.wait()

            def pair_body(i, carry2):
                e0 = 2 * i
                e1 = 2 * i + 1
                pr0 = pv[e0]
                pr1 = pv[e1]
                pe0 = jnp.where(low8, pr0[h0], pr0[h0 + 1])
                pe1 = jnp.where(low8, pr1[h0], pr1[h0 + 1])
                cb[e0] = cb[e0] * pe0
                cb[e1] = cb[e1] * pe1
                return carry2
            lax.fori_loop(0, NPAIR, pair_body, 0)

            cps = []
            for k in range(NSUBBLK):
                cps.append(pltpu.async_copy(
                    cb.at[pl.ds(k * SUB, SUB)], s2.at[didx.at[k]], sem,
                    add=True))
            for cp in cps:
                cp.wait()
            return carry
        lax.fori_loop(0, nit, iter_body, 0)

        plsc.subcore_barrier()

        def mk_out(L, g=g):
            def f():
                pltpu.sync_copy(s2.at[pl.ds(s_lo, L)],
                                spart_ref.at[g, pl.ds(cid * N + s_lo, L)])
            return f

        pl.when(sid < 15)(mk_out(CHUNK))
        pl.when(sid == 15)(mk_out(NSL))


def _edge_b(dst2d, p_flat, c_flat, init4, zin):
    k = pl.kernel(
        _edge_b_body,
        out_type=[jax.ShapeDtypeStruct((4, NCORES * N, 16), jnp.float32)],
        mesh=_SC_MESH,
        compiler_params=_SC_PARAMS,
        scratch_types=[
            pltpu.VMEM((NSUBBLK, SUB), jnp.int32),
            pltpu.VMEM((BLK, 16), jnp.float32),
            pltpu.VMEM((BLK, 16), jnp.float32),
            pltpu.VMEM_SHARED((N, 16), jnp.float32),
            pltpu.SemaphoreType.DMA,
        ],
    )
    (spart,) = k(dst2d, p_flat, c_flat, init4, zin)
    return spart


def _comb_body(sp_ref, wh_ref, b_ref, out_ref):
    sp = sp_ref[...]
    s = sp[:, 0] + sp[:, 1]
    bn = s.shape[1]
    acc = jnp.zeros((bn, NCH), jnp.float32)
    for g in range(4):
        for j in range(2):
            sh = s[g, :, 8 * j:8 * j + 8]
            r = sh[:, 1:8] / (sh[:, 0:1] + 1e-16)
            acc = acc + r @ wh_ref[2 * g + j]
    o = acc * (1.0 / NH) + b_ref[...][None, :]
    out_ref[...] = jnp.where(o > 0, o, jnp.exp(o) - 1.0)


def _combine(spart, wh_t, bias):
    BN = 1000
    g = N // BN
    return pl.pallas_call(
        _comb_body,
        grid=(g,),
        in_specs=[
            pl.BlockSpec((4, NCORES, BN, 16), lambda i: (0, 0, i, 0)),
            pl.BlockSpec((NH, 7, NCH), lambda i: (0, 0, 0)),
            pl.BlockSpec((NCH,), lambda i: (0,)),
        ],
        out_specs=pl.BlockSpec((BN, NCH), lambda i: (i, 0)),
        out_shape=jax.ShapeDtypeStruct((N, NCH), jnp.float32),
    )(spart, wh_t, bias)


def _pair_body(cu_ref, cv_ref, h_ref, u_ref, v_ref, iu, iv, bu, bv, sem):
    wid = lax.axis_index("s") * NCORES + lax.axis_index("c")

    def iter_body(j, carry):
        base = wid * CPT + j * 128
        c1 = pltpu.async_copy(cu_ref.at[pl.ds(base, 128)], iu, sem)
        c2 = pltpu.async_copy(cv_ref.at[pl.ds(base, 128)], iv, sem)
        c1.wait()
        c2.wait()
        a = pltpu.async_copy(h_ref.at[iu], bu, sem)
        b = pltpu.async_copy(h_ref.at[iv], bv, sem)
        a.wait()
        b.wait()
        o1 = pltpu.async_copy(bu, u_ref.at[pl.ds(base, 128)], sem)
        o2 = pltpu.async_copy(bv, v_ref.at[pl.ds(base, 128)], sem)
        o1.wait()
        o2.wait()
        return carry
    lax.fori_loop(0, CIT, iter_body, 0)


def _pair_gather(cu, cv, hemb):
    k = pl.kernel(
        _pair_body,
        out_type=[
            jax.ShapeDtypeStruct((NCP, NCH), jnp.float32),
            jax.ShapeDtypeStruct((NCP, NCH), jnp.float32),
        ],
        mesh=_SC_MESH,
        compiler_params=_SC_PARAMS,
        scratch_types=[
            pltpu.VMEM((128,), jnp.int32),
            pltpu.VMEM((128,), jnp.int32),
            pltpu.VMEM((128, NCH), jnp.float32),
            pltpu.VMEM((128, NCH), jnp.float32),
            pltpu.SemaphoreType.DMA,
        ],
    )
    return k(cu, cv, hemb)


def _mlp_body(u_ref, v_ref, w1_ref, b1_ref, w2_ref, b2_ref, out_ref):
    u = u_ref[...]
    v = v_ref[...]
    pair = jnp.concatenate([u, v, jnp.abs(u - v)], axis=1)
    hid = jnp.maximum(pair @ w1_ref[...] + b1_ref[...][None, :], 0.0)
    logit = hid @ w2_ref[...] + b2_ref[...][None, :]
    out_ref[...] = jax.nn.sigmoid(logit)


def _mlp(u, v, W1, b1, W2, b2):
    BN = 5000
    g = NCAND // BN
    out2d = pl.pallas_call(
        _mlp_body,
        grid=(g,),
        in_specs=[
            pl.BlockSpec((BN, NCH), lambda i: (i, 0)),
            pl.BlockSpec((BN, NCH), lambda i: (i, 0)),
            pl.BlockSpec((96, 128), lambda i: (0, 0)),
            pl.BlockSpec((128,), lambda i: (0,)),
            pl.BlockSpec((128, 1), lambda i: (0, 0)),
            pl.BlockSpec((1,), lambda i: (0,)),
        ],
        out_specs=pl.BlockSpec((BN, 1), lambda i: (i, 0)),
        out_shape=jax.ShapeDtypeStruct((NCAND, 1), jnp.float32),
    )(u, v, W1, b1, W2, b2)
    return out2d.reshape(NCAND)


def kernel(x, edge_index, cand_edges, W, att_src, att_dst, bias, W1, b1, W2, b2):
    D = x.shape[1]
    wh = W.reshape(D, NH, NCH)
    a_s = jnp.einsum("dhc,hc->dh", wh, att_src)
    a_d = jnp.einsum("dhc,hc->dh", wh, att_dst)
    aa = jnp.concatenate([a_s, a_d], axis=1)
    wh_t = wh.transpose(1, 0, 2)

    as2, ad2, xa, init_s = _prep(x, aa)
    init4 = init_s.transpose(1, 0, 2).reshape(4, 2, N, 8)
    init4 = init4.transpose(0, 2, 1, 3).reshape(4, N, 16)

    src2d = edge_index[0].reshape(E // SUB, SUB)
    dst2d = edge_index[1].reshape(E // SUB, SUB)
    zin = jnp.zeros((N, 16), jnp.float32)

    p_arr, c_arr = _edge_a(src2d, dst2d, as2, ad2, xa)
    spart = _edge_b(dst2d, p_arr, c_arr, init4, zin)
    hemb = _combine(spart.reshape(4, NCORES, N, 16), wh_t, bias)

    cu = jnp.pad(cand_edges[:, 0], (0, NCP - NCAND))
    cv = jnp.pad(cand_edges[:, 1], (0, NCP - NCAND))
    u, v = _pair_gather(cu, cv, hemb)
    return _mlp(u[:NCAND], v[:NCAND], W1, b1, W2, b2)


# speedup vs baseline: 112.6240x; 1.0215x over previous
import functools

import jax
import jax.numpy as jnp
from jax import lax
from jax.experimental import pallas as pl
from jax.experimental.pallas import tpu as pltpu, tpu_sc as plsc

N = 100000
E = 3200000
NCAND = 500000
NH = 8
NCH = 32
NCORES = 2
NSUB = 16
NW = NCORES * NSUB

SUB = 80
NSUBBLK = 8
BLK = SUB * NSUBBLK
NBTOT = E // BLK
NPAIR = BLK // 2

BLKA = SUB * 16
NBTOTA = E // BLKA
NPAIRA = BLKA // 2

NCP = NW * 123 * 128
CPT = NCP // NW
CIT = CPT // 128

_SC_MESH = plsc.VectorSubcoreMesh(core_axis_name="c", subcore_axis_name="s")
_SC_PARAMS = pltpu.CompilerParams(use_tc_tiling_on_sc=False)


def _prep_body(x_ref, aa_ref, as2_ref, ad2_ref, xa_ref, init_ref):
    xb = x_ref[...]
    asd = xb @ aa_ref[...]
    a_s = asd[:, :8]
    a_d = asd[:, 8:]
    as2_ref[...] = jnp.concatenate([a_s, a_s], axis=1)
    ad2_ref[...] = jnp.concatenate([a_d, a_d], axis=1)
    bn = xb.shape[0]
    xab = jnp.concatenate([jnp.ones((bn, 1), jnp.float32), xb], axis=1)
    xa_ref[...] = jnp.concatenate([xab, xab], axis=1)
    al = a_s + a_d
    al = jnp.where(al >= 0, al, 0.2 * al)
    ps = jnp.exp(al)
    init_ref[...] = ps[:, :, None] * xab[:, None, :]


def _prep(x, aa):
    BN = 1000
    g = N // BN
    return pl.pallas_call(
        _prep_body,
        grid=(g,),
        in_specs=[
            pl.BlockSpec((BN, 7), lambda i: (i, 0)),
            pl.BlockSpec((7, 16), lambda i: (0, 0)),
        ],
        out_specs=[
            pl.BlockSpec((BN, 16), lambda i: (i, 0)),
            pl.BlockSpec((BN, 16), lambda i: (i, 0)),
            pl.BlockSpec((BN, 16), lambda i: (i, 0)),
            pl.BlockSpec((BN, 8, 8), lambda i: (i, 0, 0)),
        ],
        out_shape=[
            jax.ShapeDtypeStruct((N, 16), jnp.float32),
            jax.ShapeDtypeStruct((N, 16), jnp.float32),
            jax.ShapeDtypeStruct((N, 16), jnp.float32),
            jax.ShapeDtypeStruct((N, 8, 8), jnp.float32),
        ],
    )(x, aa)


def _edge_a_body(src_ref, dst_ref, as2_ref, ad2_ref, xa_ref, p_ref, c_ref,
                 sidx, didx, sa, da, cg, pb, sem):
    wid = lax.axis_index("s") * NCORES + lax.axis_index("c")

    def iter_body(j, carry):
        b = j * NW + wid
        rowb = b * 16
        base = b * BLKA
        c1 = pltpu.async_copy(src_ref.at[pl.ds(rowb, 16)], sidx, sem)
        c2 = pltpu.async_copy(dst_ref.at[pl.ds(rowb, 16)], didx, sem)
        c1.wait()
        c2.wait()
        cps = []
        for k in range(16):
            cps.append(pltpu.async_copy(
                as2_ref.at[sidx.at[k]], sa.at[pl.ds(k * SUB, SUB)], sem))
            cps.append(pltpu.async_copy(
                ad2_ref.at[didx.at[k]], da.at[pl.ds(k * SUB, SUB)], sem))
            cps.append(pltpu.async_copy(
                xa_ref.at[sidx.at[k]], cg.at[pl.ds(k * SUB, SUB)], sem))
        for cp in cps:
            cp.wait()

        def pair_body(i, carry2):
            e0 = 2 * i
            v0 = sa[e0] + da[e0]
            v0 = jnp.where(v0 >= 0, v0, 0.2 * v0)
            pb[e0] = jnp.exp(v0)
            e1 = 2 * i + 1
            v1 = sa[e1] + da[e1]
            v1 = jnp.where(v1 >= 0, v1, 0.2 * v1)
            pb[e1] = jnp.exp(v1)
            return carry2
        lax.fori_loop(0, NPAIRA, pair_body, 0)

        o1 = pltpu.async_copy(pb, p_ref.at[pl.ds(base, BLKA)], sem)
        o2 = pltpu.async_copy(cg, c_ref.at[pl.ds(base, BLKA)], sem)
        o1.wait()
        o2.wait()
        return carry
    nit = (NBTOTA // NW) + jnp.where(wid < NBTOTA % NW, 1, 0)
    lax.fori_loop(0, nit, iter_body, 0)


def _edge_a(src2d, dst2d, as2, ad2, xa):
    k = pl.kernel(
        _edge_a_body,
        out_type=[
            jax.ShapeDtypeStruct((E, 16), jnp.float32),
            jax.ShapeDtypeStruct((E, 16), jnp.float32),
        ],
        mesh=_SC_MESH,
        compiler_params=_SC_PARAMS,
        scratch_types=[
            pltpu.VMEM((16, SUB), jnp.int32),
            pltpu.VMEM((16, SUB), jnp.int32),
            pltpu.VMEM((BLKA, 16), jnp.float32),
            pltpu.VMEM((BLKA, 16), jnp.float32),
            pltpu.VMEM((BLKA, 16), jnp.float32),
            pltpu.VMEM((BLKA, 16), jnp.float32),
            pltpu.SemaphoreType.DMA,
        ],
    )
    return k(src2d, dst2d, as2, ad2, xa)


def _edge_b_body(dst_ref, p_ref, c_ref, init_ref, zin_ref, spart_ref,
                 didx, pv, cb, s2, sem):
    cid = lax.axis_index("c")
    sid = lax.axis_index("s")
    wid = sid * NCORES + cid
    low8 = lax.iota(jnp.int32, 16) < 8
    CHUNK = 6256
    s_lo = sid * CHUNK
    NSL = N - 15 * CHUNK
    nit = (NBTOT // NW) + jnp.where(wid < NBTOT % NW, 1, 0)

    for g in range(4):
        h0 = 2 * g

        def mk_init(L, from_init, g=g):
            def f():
                if from_init:
                    pltpu.sync_copy(init_ref.at[g, pl.ds(s_lo, L)],
                                    s2.at[pl.ds(s_lo, L)])
                else:
                    pltpu.sync_copy(zin_ref.at[pl.ds(s_lo, L)],
                                    s2.at[pl.ds(s_lo, L)])
            return f

        pl.when((cid == 0) & (sid < 15))(mk_init(CHUNK, True))
        pl.when((cid == 0) & (sid == 15))(mk_init(NSL, True))
        pl.when((cid == 1) & (sid < 15))(mk_init(CHUNK, False))
        pl.when((cid == 1) & (sid == 15))(mk_init(NSL, False))

        plsc.subcore_barrier()

        def iter_body(j, carry, h0=h0):
            b = j * NW + wid
            rowb = b * NSUBBLK
            base = b * BLK
            c1 = pltpu.async_copy(dst_ref.at[pl.ds(rowb, NSUBBLK)], didx, sem)
            c2 = pltpu.async_copy(c_ref.at[pl.ds(base, BLK)], cb, sem)
            c3 = pltpu.async_copy(p_ref.at[pl.ds(base, BLK)], pv, sem)
            c1.wait()
            c2.wait()
            c3.wait()

            def pair_body(i, carry2):
                e0 = 2 * i
                e1 = 2 * i + 1
                pr0 = pv[e0]
                pr1 = pv[e1]
                pe0 = jnp.where(low8, pr0[h0], pr0[h0 + 1])
                pe1 = jnp.where(low8, pr1[h0], pr1[h0 + 1])
                cb[e0] = cb[e0] * pe0
                cb[e1] = cb[e1] * pe1
                return carry2
            lax.fori_loop(0, NPAIR, pair_body, 0)

            cps = []
            for k in range(NSUBBLK):
                cps.append(pltpu.async_copy(
                    cb.at[pl.ds(k * SUB, SUB)], s2.at[didx.at[k]], sem,
                    add=True))
            for cp in cps:
                cp.wait()
            return carry
        lax.fori_loop(0, nit, iter_body, 0)

        plsc.subcore_barrier()

        def mk_out(L, g=g):
            def f():
                pltpu.sync_copy(s2.at[pl.ds(s_lo, L)],
                                spart_ref.at[g, pl.ds(cid * N + s_lo, L)])
            return f

        pl.when(sid < 15)(mk_out(CHUNK))
        pl.when(sid == 15)(mk_out(NSL))


def _edge_b(dst2d, p_flat, c_flat, init4, zin):
    k = pl.kernel(
        _edge_b_body,
        out_type=[jax.ShapeDtypeStruct((4, NCORES * N, 16), jnp.float32)],
        mesh=_SC_MESH,
        compiler_params=_SC_PARAMS,
        scratch_types=[
            pltpu.VMEM((NSUBBLK, SUB), jnp.int32),
            pltpu.VMEM((BLK, 16), jnp.float32),
            pltpu.VMEM((BLK, 16), jnp.float32),
            pltpu.VMEM_SHARED((N, 16), jnp.float32),
            pltpu.SemaphoreType.DMA,
        ],
    )
    (spart,) = k(dst2d, p_flat, c_flat, init4, zin)
    return spart


def _comb_body(sp_ref, wh_ref, b_ref, out_ref):
    sp = sp_ref[...]
    s = sp[:, 0] + sp[:, 1]
    bn = s.shape[1]
    acc = jnp.zeros((bn, NCH), jnp.float32)
    for g in range(4):
        for j in range(2):
            sh = s[g, :, 8 * j:8 * j + 8]
            r = sh[:, 1:8] / (sh[:, 0:1] + 1e-16)
            acc = acc + r @ wh_ref[2 * g + j]
    o = acc * (1.0 / NH) + b_ref[...][None, :]
    out_ref[...] = jnp.where(o > 0, o, jnp.exp(o) - 1.0)


def _combine(spart, wh_t, bias):
    BN = 1000
    g = N // BN
    return pl.pallas_call(
        _comb_body,
        grid=(g,),
        in_specs=[
            pl.BlockSpec((4, NCORES, BN, 16), lambda i: (0, 0, i, 0)),
            pl.BlockSpec((NH, 7, NCH), lambda i: (0, 0, 0)),
            pl.BlockSpec((NCH,), lambda i: (0,)),
        ],
        out_specs=pl.BlockSpec((BN, NCH), lambda i: (i, 0)),
        out_shape=jax.ShapeDtypeStruct((N, NCH), jnp.float32),
    )(spart, wh_t, bias)


def _pair_body(cu_ref, cv_ref, h_ref, u_ref, v_ref, iu, iv, bu, bv, sem):
    wid = lax.axis_index("s") * NCORES + lax.axis_index("c")

    def iter_body(j, carry):
        base = wid * CPT + j * 128
        c1 = pltpu.async_copy(cu_ref.at[pl.ds(base, 128)], iu, sem)
        c2 = pltpu.async_copy(cv_ref.at[pl.ds(base, 128)], iv, sem)
        c1.wait()
        c2.wait()
        a = pltpu.async_copy(h_ref.at[iu], bu, sem)
        b = pltpu.async_copy(h_ref.at[iv], bv, sem)
        a.wait()
        b.wait()
        o1 = pltpu.async_copy(bu, u_ref.at[pl.ds(base, 128)], sem)
        o2 = pltpu.async_copy(bv, v_ref.at[pl.ds(base, 128)], sem)
        o1.wait()
        o2.wait()
        return carry
    lax.fori_loop(0, CIT, iter_body, 0)


def _pair_gather(cu, cv, hemb):
    k = pl.kernel(
        _pair_body,
        out_type=[
            jax.ShapeDtypeStruct((NCP, NCH), jnp.float32),
            jax.ShapeDtypeStruct((NCP, NCH), jnp.float32),
        ],
        mesh=_SC_MESH,
        compiler_params=_SC_PARAMS,
        scratch_types=[
            pltpu.VMEM((128,), jnp.int32),
            pltpu.VMEM((128,), jnp.int32),
            pltpu.VMEM((128, NCH), jnp.float32),
            pltpu.VMEM((128, NCH), jnp.float32),
            pltpu.SemaphoreType.DMA,
        ],
    )
    return k(cu, cv, hemb)


def _mlp_body(u_ref, v_ref, w1_ref, b1_ref, w2_ref, b2_ref, out_ref):
    u = u_ref[...]
    v = v_ref[...]
    pair = jnp.concatenate([u, v, jnp.abs(u - v)], axis=1)
    hid = jnp.maximum(pair @ w1_ref[...] + b1_ref[...][None, :], 0.0)
    logit = hid @ w2_ref[...] + b2_ref[...][None, :]
    out_ref[...] = jax.nn.sigmoid(logit)


def _mlp(u, v, W1, b1, W2, b2):
    BN = 5000
    g = NCAND // BN
    out2d = pl.pallas_call(
        _mlp_body,
        grid=(g,),
        in_specs=[
            pl.BlockSpec((BN, NCH), lambda i: (i, 0)),
            pl.BlockSpec((BN, NCH), lambda i: (i, 0)),
            pl.BlockSpec((96, 128), lambda i: (0, 0)),
            pl.BlockSpec((128,), lambda i: (0,)),
            pl.BlockSpec((128, 1), lambda i: (0, 0)),
            pl.BlockSpec((1,), lambda i: (0,)),
        ],
        out_specs=pl.BlockSpec((BN, 1), lambda i: (i, 0)),
        out_shape=jax.ShapeDtypeStruct((NCAND, 1), jnp.float32),
    )(u, v, W1, b1, W2, b2)
    return out2d.reshape(NCAND)


def kernel(x, edge_index, cand_edges, W, att_src, att_dst, bias, W1, b1, W2, b2):
    D = x.shape[1]
    wh = W.reshape(D, NH, NCH)
    a_s = jnp.einsum("dhc,hc->dh", wh, att_src)
    a_d = jnp.einsum("dhc,hc->dh", wh, att_dst)
    aa = jnp.concatenate([a_s, a_d], axis=1)
    wh_t = wh.transpose(1, 0, 2)

    as2, ad2, xa, init_s = _prep(x, aa)
    init4 = init_s.transpose(1, 0, 2).reshape(4, 2, N, 8)
    init4 = init4.transpose(0, 2, 1, 3).reshape(4, N, 16)

    src2d = edge_index[0].reshape(E // SUB, SUB)
    dst2d = edge_index[1].reshape(E // SUB, SUB)
    zin = jnp.zeros((N, 16), jnp.float32)

    p_arr, c_arr = _edge_a(src2d, dst2d, as2, ad2, xa)
    spart = _edge_b(dst2d, p_arr, c_arr, init4, zin)
    hemb = _combine(spart.reshape(4, NCORES, N, 16), wh_t, bias)

    cu = jnp.pad(cand_edges[:, 0], (0, NCP - NCAND))
    cv = jnp.pad(cand_edges[:, 1], (0, NCP - NCAND))
    u, v = _pair_gather(cu, cv, hemb)
    return _mlp(u[:NCAND], v[:NCAND], W1, b1, W2, b2)
